# Initial kernel scaffold; baseline (speedup 1.0000x reference)
import jax
import jax.numpy as jnp
from jax.experimental import pallas as pl


def kernel(timestep, log_sigmas):
    raise NotImplementedError("write your pallas kernel here")


# speedup vs baseline: 530.8312x; 530.8312x over previous
import functools

import jax
import jax.numpy as jnp
from jax import lax
from jax.experimental import pallas as pl
from jax.experimental.pallas import tpu as pltpu
from jax.experimental.pallas import tpu_sc as plsc

_N = 1048576
_NTAB = 1000
_TAB_PAD = 1008
_LANES = 16
_CHUNK = 4096


@functools.lru_cache(maxsize=None)
def _build_kernel():
    info = plsc.get_sparse_core_info()
    num_cores = info.num_cores
    num_subcores = info.num_subcores
    num_workers = num_cores * num_subcores
    per_worker = _N // num_workers
    n_chunks = per_worker // _CHUNK

    mesh = plsc.VectorSubcoreMesh(core_axis_name="c", subcore_axis_name="s")

    @functools.partial(
        pl.kernel,
        mesh=mesh,
        out_type=jax.ShapeDtypeStruct((_N,), jnp.float32),
        compiler_params=pltpu.CompilerParams(needs_layout_passes=False),
        scratch_types=[
            pltpu.VMEM((_TAB_PAD,), jnp.float32),
            pltpu.VMEM((_TAB_PAD,), jnp.float32),
            pltpu.VMEM((2, _CHUNK), jnp.int32),
            pltpu.VMEM((2, _CHUNK), jnp.float32),
            pltpu.SemaphoreType.DMA,
            pltpu.SemaphoreType.DMA,
            pltpu.SemaphoreType.DMA,
            pltpu.SemaphoreType.DMA,
        ],
    )
    def sigma_kernel(ts_hbm, ls_hbm, out_hbm, logt_v, sigt_v, idx_v, res_v,
                     in_sem0, in_sem1, out_sem0, out_sem1):
        wid = lax.axis_index("s") * num_cores + lax.axis_index("c")
        base = wid * per_worker

        tab_copy = pltpu.make_async_copy(
            ls_hbm, logt_v.at[pl.ds(0, _NTAB)], in_sem1)
        tab_copy.start()
        first_idx = pltpu.make_async_copy(
            ts_hbm.at[pl.ds(base, _CHUNK)], idx_v.at[0], in_sem0)
        first_idx.start()
        tab_copy.wait()

        def exp_body(i, carry):
            sl = pl.ds(i * _LANES, _LANES)
            sigt_v[sl] = jnp.exp(logt_v[sl])
            return carry
        lax.fori_loop(0, _TAB_PAD // _LANES, exp_body, 0)

        in_sems = (in_sem0, in_sem1)
        out_sems = (out_sem0, out_sem1)

        for c in range(n_chunks):
            buf = c % 2
            if c + 1 < n_chunks:
                nxt = base + (c + 1) * _CHUNK
                pltpu.make_async_copy(
                    ts_hbm.at[pl.ds(nxt, _CHUNK)],
                    idx_v.at[1 - buf],
                    in_sems[1 - buf],
                ).start()
            pltpu.make_async_copy(
                ts_hbm.at[pl.ds(base + c * _CHUNK, _CHUNK)],
                idx_v.at[buf],
                in_sems[buf],
            ).wait()
            if c >= 2:
                pltpu.make_async_copy(
                    res_v.at[buf],
                    out_hbm.at[pl.ds(base + (c - 2) * _CHUNK, _CHUNK)],
                    out_sems[buf],
                ).wait()

            def gather_body(j, carry, _buf=buf):
                sl = pl.ds(j * _LANES, _LANES)
                idx = idx_v[_buf, sl]
                res_v[_buf, sl] = plsc.load_gather(sigt_v, [idx])
                return carry
            lax.fori_loop(0, _CHUNK // _LANES, gather_body, 0)

            pltpu.make_async_copy(
                res_v.at[buf],
                out_hbm.at[pl.ds(base + c * _CHUNK, _CHUNK)],
                out_sems[buf],
            ).start()

        for c in (n_chunks - 2, n_chunks - 1):
            pltpu.make_async_copy(
                res_v.at[c % 2],
                out_hbm.at[pl.ds(base + c * _CHUNK, _CHUNK)],
                out_sems[c % 2],
            ).wait()

    return sigma_kernel


def kernel(timestep, log_sigmas):
    return _build_kernel()(timestep, log_sigmas)


# speedup vs baseline: 681.5317x; 1.2839x over previous
import functools

import jax
import jax.numpy as jnp
from jax import lax
from jax.experimental import pallas as pl
from jax.experimental.pallas import tpu as pltpu
from jax.experimental.pallas import tpu_sc as plsc

_N = 1048576
_NTAB = 1000
_TAB_PAD = 1008
_LANES = 16
_CHUNK = 4096


@functools.lru_cache(maxsize=None)
def _build_kernel():
    info = plsc.get_sparse_core_info()
    num_cores = info.num_cores
    num_subcores = info.num_subcores
    num_workers = num_cores * num_subcores
    per_worker = _N // num_workers
    n_chunks = per_worker // _CHUNK

    mesh = plsc.VectorSubcoreMesh(core_axis_name="c", subcore_axis_name="s")

    @functools.partial(
        pl.kernel,
        mesh=mesh,
        out_type=jax.ShapeDtypeStruct((_N,), jnp.float32),
        compiler_params=pltpu.CompilerParams(needs_layout_passes=False),
        scratch_types=[
            pltpu.VMEM((_TAB_PAD,), jnp.float32),
            pltpu.VMEM((_TAB_PAD,), jnp.float32),
            pltpu.VMEM((2, _CHUNK), jnp.int32),
            pltpu.VMEM((2, _CHUNK), jnp.float32),
            pltpu.SemaphoreType.DMA,
            pltpu.SemaphoreType.DMA,
            pltpu.SemaphoreType.DMA,
            pltpu.SemaphoreType.DMA,
        ],
    )
    def sigma_kernel(ts_hbm, ls_hbm, out_hbm, logt_v, sigt_v, idx_v, res_v,
                     in_sem0, in_sem1, out_sem0, out_sem1):
        wid = lax.axis_index("s") * num_cores + lax.axis_index("c")
        base = wid * per_worker

        tab_copy = pltpu.make_async_copy(
            ls_hbm, logt_v.at[pl.ds(0, _NTAB)], in_sem1)
        tab_copy.start()
        first_idx = pltpu.make_async_copy(
            ts_hbm.at[pl.ds(base, _CHUNK)], idx_v.at[0], in_sem0)
        first_idx.start()
        tab_copy.wait()

        @plsc.parallel_loop(0, _TAB_PAD, _LANES, unroll=4)
        def exp_body(i):
            sl = pl.ds(i, _LANES)
            sigt_v[sl] = jnp.exp(logt_v[sl])

        in_sems = (in_sem0, in_sem1)
        out_sems = (out_sem0, out_sem1)

        for c in range(n_chunks):
            buf = c % 2
            if c + 1 < n_chunks:
                nxt = base + (c + 1) * _CHUNK
                pltpu.make_async_copy(
                    ts_hbm.at[pl.ds(nxt, _CHUNK)],
                    idx_v.at[1 - buf],
                    in_sems[1 - buf],
                ).start()
            pltpu.make_async_copy(
                ts_hbm.at[pl.ds(base + c * _CHUNK, _CHUNK)],
                idx_v.at[buf],
                in_sems[buf],
            ).wait()
            if c >= 2:
                pltpu.make_async_copy(
                    res_v.at[buf],
                    out_hbm.at[pl.ds(base + (c - 2) * _CHUNK, _CHUNK)],
                    out_sems[buf],
                ).wait()

            @plsc.parallel_loop(0, _CHUNK, _LANES, unroll=8)
            def gather_body(j, _buf=buf):
                sl = pl.ds(j, _LANES)
                idx = idx_v[_buf, sl]
                res_v[_buf, sl] = plsc.load_gather(sigt_v, [idx])

            pltpu.make_async_copy(
                res_v.at[buf],
                out_hbm.at[pl.ds(base + c * _CHUNK, _CHUNK)],
                out_sems[buf],
            ).start()

        for c in (n_chunks - 2, n_chunks - 1):
            pltpu.make_async_copy(
                res_v.at[c % 2],
                out_hbm.at[pl.ds(base + c * _CHUNK, _CHUNK)],
                out_sems[c % 2],
            ).wait()

    return sigma_kernel


def kernel(timestep, log_sigmas):
    return _build_kernel()(timestep, log_sigmas)


# speedup vs baseline: 710.4860x; 1.0425x over previous
import functools

import jax
import jax.numpy as jnp
from jax import lax
from jax.experimental import pallas as pl
from jax.experimental.pallas import tpu as pltpu
from jax.experimental.pallas import tpu_sc as plsc

_N = 1048576
_NTAB = 1000
_TAB_PAD = 1008
_LANES = 16
_CHUNK = 8192


@functools.lru_cache(maxsize=None)
def _build_kernel():
    info = plsc.get_sparse_core_info()
    num_cores = info.num_cores
    num_subcores = info.num_subcores
    num_workers = num_cores * num_subcores
    per_worker = _N // num_workers
    n_chunks = per_worker // _CHUNK

    mesh = plsc.VectorSubcoreMesh(core_axis_name="c", subcore_axis_name="s")

    @functools.partial(
        pl.kernel,
        mesh=mesh,
        out_type=jax.ShapeDtypeStruct((_N,), jnp.float32),
        compiler_params=pltpu.CompilerParams(needs_layout_passes=False),
        scratch_types=[
            pltpu.VMEM((_TAB_PAD,), jnp.float32),
            pltpu.VMEM((_TAB_PAD,), jnp.float32),
            pltpu.VMEM((2, _CHUNK), jnp.int32),
            pltpu.VMEM((2, _CHUNK), jnp.float32),
            pltpu.SemaphoreType.DMA,
            pltpu.SemaphoreType.DMA,
            pltpu.SemaphoreType.DMA,
            pltpu.SemaphoreType.DMA,
        ],
    )
    def sigma_kernel(ts_hbm, ls_hbm, out_hbm, logt_v, sigt_v, idx_v, res_v,
                     in_sem0, in_sem1, out_sem0, out_sem1):
        wid = lax.axis_index("s") * num_cores + lax.axis_index("c")
        base = wid * per_worker

        tab_copy = pltpu.make_async_copy(
            ls_hbm, logt_v.at[pl.ds(0, _NTAB)], in_sem1)
        tab_copy.start()
        first_idx = pltpu.make_async_copy(
            ts_hbm.at[pl.ds(base, _CHUNK)], idx_v.at[0], in_sem0)
        first_idx.start()
        tab_copy.wait()

        @plsc.parallel_loop(0, _TAB_PAD, _LANES, unroll=4)
        def exp_body(i):
            sl = pl.ds(i, _LANES)
            sigt_v[sl] = jnp.exp(logt_v[sl])

        in_sems = (in_sem0, in_sem1)
        out_sems = (out_sem0, out_sem1)

        for c in range(n_chunks):
            buf = c % 2
            if c + 1 < n_chunks:
                nxt = base + (c + 1) * _CHUNK
                pltpu.make_async_copy(
                    ts_hbm.at[pl.ds(nxt, _CHUNK)],
                    idx_v.at[1 - buf],
                    in_sems[1 - buf],
                ).start()
            pltpu.make_async_copy(
                ts_hbm.at[pl.ds(base + c * _CHUNK, _CHUNK)],
                idx_v.at[buf],
                in_sems[buf],
            ).wait()
            if c >= 2:
                pltpu.make_async_copy(
                    res_v.at[buf],
                    out_hbm.at[pl.ds(base + (c - 2) * _CHUNK, _CHUNK)],
                    out_sems[buf],
                ).wait()

            @plsc.parallel_loop(0, _CHUNK, _LANES, unroll=8)
            def gather_body(j, _buf=buf):
                sl = pl.ds(j, _LANES)
                idx = idx_v[_buf, sl]
                res_v[_buf, sl] = plsc.load_gather(sigt_v, [idx])

            pltpu.make_async_copy(
                res_v.at[buf],
                out_hbm.at[pl.ds(base + c * _CHUNK, _CHUNK)],
                out_sems[buf],
            ).start()

        for c in (n_chunks - 2, n_chunks - 1):
            pltpu.make_async_copy(
                res_v.at[c % 2],
                out_hbm.at[pl.ds(base + c * _CHUNK, _CHUNK)],
                out_sems[c % 2],
            ).wait()

    return sigma_kernel


def kernel(timestep, log_sigmas):
    return _build_kernel()(timestep, log_sigmas)


# speedup vs baseline: 719.6521x; 1.0129x over previous
import functools

import jax
import jax.numpy as jnp
from jax import lax
from jax.experimental import pallas as pl
from jax.experimental.pallas import tpu as pltpu
from jax.experimental.pallas import tpu_sc as plsc

_N = 1048576
_NTAB = 1000
_TAB_PAD = 1008
_LANES = 16
_CHUNK = 16384


@functools.lru_cache(maxsize=None)
def _build_kernel():
    info = plsc.get_sparse_core_info()
    num_cores = info.num_cores
    num_subcores = info.num_subcores
    num_workers = num_cores * num_subcores
    per_worker = _N // num_workers
    n_chunks = per_worker // _CHUNK

    mesh = plsc.VectorSubcoreMesh(core_axis_name="c", subcore_axis_name="s")

    @functools.partial(
        pl.kernel,
        mesh=mesh,
        out_type=jax.ShapeDtypeStruct((_N,), jnp.float32),
        compiler_params=pltpu.CompilerParams(needs_layout_passes=False),
        scratch_types=[
            pltpu.VMEM((_TAB_PAD,), jnp.float32),
            pltpu.VMEM((_TAB_PAD,), jnp.float32),
            pltpu.VMEM((2, _CHUNK), jnp.int32),
            pltpu.VMEM((2, _CHUNK), jnp.float32),
            pltpu.SemaphoreType.DMA,
            pltpu.SemaphoreType.DMA,
            pltpu.SemaphoreType.DMA,
            pltpu.SemaphoreType.DMA,
        ],
    )
    def sigma_kernel(ts_hbm, ls_hbm, out_hbm, logt_v, sigt_v, idx_v, res_v,
                     in_sem0, in_sem1, out_sem0, out_sem1):
        wid = lax.axis_index("s") * num_cores + lax.axis_index("c")
        base = wid * per_worker

        tab_copy = pltpu.make_async_copy(
            ls_hbm, logt_v.at[pl.ds(0, _NTAB)], in_sem1)
        tab_copy.start()
        first_idx = pltpu.make_async_copy(
            ts_hbm.at[pl.ds(base, _CHUNK)], idx_v.at[0], in_sem0)
        first_idx.start()
        tab_copy.wait()

        @plsc.parallel_loop(0, _TAB_PAD, _LANES, unroll=4)
        def exp_body(i):
            sl = pl.ds(i, _LANES)
            sigt_v[sl] = jnp.exp(logt_v[sl])

        in_sems = (in_sem0, in_sem1)
        out_sems = (out_sem0, out_sem1)

        for c in range(n_chunks):
            buf = c % 2
            if c + 1 < n_chunks:
                nxt = base + (c + 1) * _CHUNK
                pltpu.make_async_copy(
                    ts_hbm.at[pl.ds(nxt, _CHUNK)],
                    idx_v.at[1 - buf],
                    in_sems[1 - buf],
                ).start()
            pltpu.make_async_copy(
                ts_hbm.at[pl.ds(base + c * _CHUNK, _CHUNK)],
                idx_v.at[buf],
                in_sems[buf],
            ).wait()
            if c >= 2:
                pltpu.make_async_copy(
                    res_v.at[buf],
                    out_hbm.at[pl.ds(base + (c - 2) * _CHUNK, _CHUNK)],
                    out_sems[buf],
                ).wait()

            @plsc.parallel_loop(0, _CHUNK, _LANES, unroll=8)
            def gather_body(j, _buf=buf):
                sl = pl.ds(j, _LANES)
                idx = idx_v[_buf, sl]
                res_v[_buf, sl] = plsc.load_gather(sigt_v, [idx])

            pltpu.make_async_copy(
                res_v.at[buf],
                out_hbm.at[pl.ds(base + c * _CHUNK, _CHUNK)],
                out_sems[buf],
            ).start()

        for c in (n_chunks - 2, n_chunks - 1):
            pltpu.make_async_copy(
                res_v.at[c % 2],
                out_hbm.at[pl.ds(base + c * _CHUNK, _CHUNK)],
                out_sems[c % 2],
            ).wait()

    return sigma_kernel


def kernel(timestep, log_sigmas):
    return _build_kernel()(timestep, log_sigmas)
